# baseline (speedup 1.0000x reference)
import functools

import jax
import jax.numpy as jnp
from jax import lax
from jax.experimental import pallas as pl
from jax.experimental.pallas import tpu as pltpu
from jax.experimental.pallas import tpu_sc as plsc

TOP_K = 2
E = 8
D = 768
N = 4096
P = 2
D2 = D * P
ROWS = N * P
BMA = 256
NPAD = ROWS + E * BMA
NB = NPAD // BMA
NBO = 48

NW = 32
TPW = N // NW

_MESH = plsc.VectorSubcoreMesh(
    core_axis_name="c", subcore_axis_name="s", num_cores=2, num_subcores=16
)


def _wid():
    return lax.axis_index("s") * 2 + lax.axis_index("c")


def _router_body(aff_ref, eps_ref, pa_ref, pb_ref, g1_ref, g2_ref, eob_ref):
    aff = aff_ref[...].T
    eps = eps_ref[...].T
    sp = jnp.maximum(aff, 0.0) + jnp.log1p(jnp.exp(-jnp.abs(aff)))
    v = aff + eps * sp

    iota_e = lax.broadcasted_iota(jnp.int32, (E, N), 0)
    cnt = jnp.zeros((E, N), jnp.int32)
    for j in range(E):
        vj = v[j:j + 1]
        beats = jnp.logical_or(
            vj > v, jnp.logical_and(vj == v, j < iota_e))
        cnt = cnt + beats.astype(jnp.int32)
    sel = cnt < TOP_K
    selF = sel.astype(jnp.float32)

    m1 = jnp.max(v, axis=0, keepdims=True)
    ex = jnp.where(sel, jnp.exp(v - m1), 0.0)
    g = ex / jnp.sum(ex, axis=0, keepdims=True)

    m8 = (lax.broadcasted_iota(jnp.int32, (E, E), 1)
          < lax.broadcasted_iota(jnp.int32, (E, E), 0)).astype(jnp.float32)
    seen = jnp.dot(m8, selF, preferred_element_type=jnp.float32)
    first = jnp.logical_and(sel, seen == 0.0)
    second = jnp.logical_and(sel, seen == 1.0)

    u128 = (lax.broadcasted_iota(jnp.int32, (TPW, TPW), 0)
            < lax.broadcasted_iota(jnp.int32, (TPW, TPW), 1)).astype(jnp.float32)
    sel256 = selF.reshape(E * NW, TPW)
    rank = jnp.dot(sel256, u128, preferred_element_type=jnp.float32)
    rank = rank.reshape(E, N)

    counts = jnp.sum(sel256, axis=1, keepdims=True).reshape(E, NW)
    u32 = (lax.broadcasted_iota(jnp.int32, (NW, NW), 0)
           < lax.broadcasted_iota(jnp.int32, (NW, NW), 1)).astype(jnp.float32)
    off = jnp.dot(counts, u32, preferred_element_type=jnp.float32)
    totals = jnp.sum(counts, axis=1, keepdims=True)
    padded = ((totals.astype(jnp.int32) + (BMA - 1)) >> 8 << 8).astype(jnp.float32)
    s = jnp.dot(m8, padded, preferred_element_type=jnp.float32)
    base = jnp.transpose(s + off).astype(jnp.int32)

    def lanesum(x):
        return jnp.sum(x, axis=0, keepdims=True)

    firstF = first.astype(jnp.float32)
    secondF = second.astype(jnp.float32)
    e1 = lanesum(firstF * iota_e.astype(jnp.float32)).astype(jnp.int32)
    e2 = lanesum(secondF * iota_e.astype(jnp.float32)).astype(jnp.int32)
    g1 = lanesum(g * firstF)
    g2 = lanesum(g * secondF)
    r1 = lanesum(rank * firstF).astype(jnp.int32)
    r2 = lanesum(rank * secondF).astype(jnp.int32)

    e1t = e1.reshape(NW, TPW)
    e2t = e2.reshape(NW, TPW)
    b1 = jnp.zeros((NW, TPW), jnp.int32)
    b2 = jnp.zeros((NW, TPW), jnp.int32)
    for e in range(E):
        be = base[:, e:e + 1]
        b1 = jnp.where(e1t == e, be, b1)
        b2 = jnp.where(e2t == e, be, b2)
    pa_ref[...] = b1 + r1.reshape(NW, TPW)
    pb_ref[...] = b2 + r2.reshape(NW, TPW)
    g1_ref[...] = g1.reshape(NW, TPW)
    g2_ref[...] = g2.reshape(NW, TPW)

    bi = (lax.broadcasted_iota(jnp.int32, (1, NBO), 1) * BMA).astype(jnp.float32)
    eb = jnp.full((1, NBO), -1.0, jnp.float32)
    for e in range(E):
        eb = eb + (bi >= s[e:e + 1, :]).astype(jnp.float32)
    eob_ref[...] = eb.astype(jnp.int32)


def _router(aff, eps):
    return pl.pallas_call(
        _router_body,
        out_shape=(
            jax.ShapeDtypeStruct((NW, TPW), jnp.int32),
            jax.ShapeDtypeStruct((NW, TPW), jnp.int32),
            jax.ShapeDtypeStruct((NW, TPW), jnp.float32),
            jax.ShapeDtypeStruct((NW, TPW), jnp.float32),
            jax.ShapeDtypeStruct((1, NBO), jnp.int32),
        ),
    )(aff, eps)


NCHUNK = 4
CT = TPW // NCHUNK


@functools.partial(
    pl.kernel,
    out_type=jax.ShapeDtypeStruct((NPAD, D2), jnp.float32),
    mesh=_MESH,
    compiler_params=pltpu.CompilerParams(needs_layout_passes=False),
    scratch_types=[
        pltpu.VMEM((NCHUNK, CT), jnp.int32),
        pltpu.VMEM((NCHUNK, CT), jnp.int32),
        pltpu.VMEM((CT, D2), jnp.float32),
        pltpu.VMEM((CT, D2), jnp.float32),
        pltpu.SemaphoreType.DMA,
        pltpu.SemaphoreType.DMA,
        pltpu.SemaphoreType.DMA,
        pltpu.SemaphoreType.DMA,
        pltpu.SemaphoreType.DMA,
        pltpu.SemaphoreType.DMA,
    ],
)
def _dispatch(x2_hbm, pa_hbm, pb_hbm, xs2_hbm,
              pa2v, pb2v, buf0, buf1, sl0, sl1, sa0, sa1, sb0, sb1):
    wid = _wid()
    tok0 = wid * TPW
    for c in range(NCHUNK):
        pltpu.sync_copy(pa_hbm.at[wid, pl.ds(c * CT, CT)], pa2v.at[c])
        pltpu.sync_copy(pb_hbm.at[wid, pl.ds(c * CT, CT)], pb2v.at[c])
    bufs = (buf0, buf1)
    sls = (sl0, sl1)
    sas = (sa0, sa1)
    sbs = (sb0, sb1)
    ld = [None, None]
    sca = [None, None]
    scb = [None, None]
    ld[0] = pltpu.async_copy(x2_hbm.at[pl.ds(tok0, CT)], bufs[0], sls[0])
    for c in range(NCHUNK):
        b = c % 2
        nb = (c + 1) % 2
        if c + 1 < NCHUNK:
            if sca[nb] is not None:
                sca[nb].wait()
                scb[nb].wait()
            ld[nb] = pltpu.async_copy(
                x2_hbm.at[pl.ds(tok0 + (c + 1) * CT, CT)], bufs[nb], sls[nb])
        ld[b].wait()
        sca[b] = pltpu.async_copy(bufs[b], xs2_hbm.at[pa2v.at[c]], sas[b])
        scb[b] = pltpu.async_copy(bufs[b], xs2_hbm.at[pb2v.at[c]], sbs[b])
    for b in range(2):
        if sca[b] is not None:
            sca[b].wait()
            scb[b].wait()


def _gmm_body(eob_ref, xs_ref, w_ref, b_ref, out_ref):
    i = pl.program_id(0)
    e = eob_ref[i]
    x = xs_ref[...].astype(jnp.bfloat16)
    w = w_ref[e]
    bias = b_ref[e, 0][None, :]
    ya = jnp.dot(x[:, :D], w, preferred_element_type=jnp.float32) + bias
    yb = jnp.dot(x[:, D:], w, preferred_element_type=jnp.float32) + bias
    out_ref[...] = jnp.concatenate([ya, yb], axis=1)


def _gmm(eob, xs, w, b):
    grid_spec = pltpu.PrefetchScalarGridSpec(
        num_scalar_prefetch=1,
        grid=(NB,),
        in_specs=[
            pl.BlockSpec((BMA, D2), lambda i, eob: (i, 0)),
            pl.BlockSpec((E, D, D), lambda i, eob: (0, 0, 0)),
            pl.BlockSpec((E, 1, D), lambda i, eob: (0, 0, 0)),
        ],
        out_specs=pl.BlockSpec((BMA, D2), lambda i, eob: (i, 0)),
    )
    return pl.pallas_call(
        _gmm_body,
        grid_spec=grid_spec,
        out_shape=jax.ShapeDtypeStruct((NPAD, D2), jnp.float32),
    )(eob, xs, w, b.reshape(E, 1, D))


KCH = 8
KT = TPW // KCH


@functools.partial(
    pl.kernel,
    out_type=jax.ShapeDtypeStruct((N, P, D), jnp.float32),
    mesh=_MESH,
    compiler_params=pltpu.CompilerParams(needs_layout_passes=False),
    scratch_types=[
        pltpu.VMEM((KCH, KT), jnp.int32),
        pltpu.VMEM((KCH, KT), jnp.int32),
        pltpu.VMEM((TPW,), jnp.float32),
        pltpu.VMEM((TPW,), jnp.float32),
        pltpu.VMEM((KT, D2), jnp.float32),
        pltpu.VMEM((KT, D2), jnp.float32),
        pltpu.VMEM((KT, D2), jnp.float32),
        pltpu.VMEM((KT, D2), jnp.float32),
        pltpu.SemaphoreType.DMA,
        pltpu.SemaphoreType.DMA,
        pltpu.SemaphoreType.DMA,
        pltpu.SemaphoreType.DMA,
        pltpu.SemaphoreType.DMA,
        pltpu.SemaphoreType.DMA,
    ],
)
def _combine(ys2_hbm, pa_hbm, pb_hbm, g1_hbm, g2_hbm, out_hbm,
             pa2v, pb2v, g1v, g2v, a0, b0, a1, b1,
             sga0, sgb0, sga1, sgb1, so0, so1):
    wid = _wid()
    tok0 = wid * TPW
    pltpu.sync_copy(g1_hbm.at[wid], g1v)
    pltpu.sync_copy(g2_hbm.at[wid], g2v)
    for c in range(KCH):
        pltpu.sync_copy(pa_hbm.at[wid, pl.ds(c * KT, KT)], pa2v.at[c])
        pltpu.sync_copy(pb_hbm.at[wid, pl.ds(c * KT, KT)], pb2v.at[c])
    abufs = (a0, a1)
    bbufs = (b0, b1)
    sgas = (sga0, sga1)
    sgbs = (sgb0, sgb1)
    sos = (so0, so1)
    ga = [None, None]
    gb = [None, None]
    wo = [None, None]
    ga[0] = pltpu.async_copy(ys2_hbm.at[pa2v.at[0]], abufs[0], sgas[0])
    gb[0] = pltpu.async_copy(ys2_hbm.at[pb2v.at[0]], bbufs[0], sgbs[0])
    for c in range(KCH):
        b = c % 2
        nb = (c + 1) % 2
        if c + 1 < KCH:
            if wo[nb] is not None:
                wo[nb][0].wait()
                wo[nb][1].wait()
            ga[nb] = pltpu.async_copy(
                ys2_hbm.at[pa2v.at[c + 1]], abufs[nb], sgas[nb])
            gb[nb] = pltpu.async_copy(
                ys2_hbm.at[pb2v.at[c + 1]], bbufs[nb], sgbs[nb])
        ga[b].wait()
        gb[b].wait()
        A = abufs[b]
        B = bbufs[b]

        def body(l, _):
            gav = plsc.load_gather(g1v, [jnp.full((16,), c * KT, jnp.int32) + l])
            gbv = plsc.load_gather(g2v, [jnp.full((16,), c * KT, jnp.int32) + l])
            for k in range(D2 // 16):
                ks = pl.ds(k * 16, 16)
                A[l, ks] = A[l, ks] * gav + B[l, ks] * gbv
            return 0

        lax.fori_loop(0, KT, body, 0)
        osl = pl.ds(tok0 + c * KT, KT)
        wo[b] = (
            pltpu.async_copy(A.at[:, pl.ds(0, D)], out_hbm.at[osl, 0], sos[b]),
            pltpu.async_copy(A.at[:, pl.ds(D, D)], out_hbm.at[osl, 1], sos[b]),
        )
    for b in range(2):
        if wo[b] is not None:
            wo[b][0].wait()
            wo[b][1].wait()


def kernel(patch_x, patch_embedding, affinity, noise_eps, expert_W, expert_b):
    x2 = patch_x.reshape(N, D2)
    pa, pb, g1, g2, eob = _router(affinity, noise_eps)
    xs2 = _dispatch(x2, pa, pb)
    ys2 = _gmm(eob.reshape(NBO), xs2, expert_W.astype(jnp.bfloat16), expert_b)
    return _combine(ys2, pa, pb, g1, g2)


# speedup vs baseline: 1.0162x; 1.0162x over previous
import functools

import jax
import jax.numpy as jnp
from jax import lax
from jax.experimental import pallas as pl
from jax.experimental.pallas import tpu as pltpu
from jax.experimental.pallas import tpu_sc as plsc

TOP_K = 2
E = 8
D = 768
N = 4096
P = 2
D2 = D * P
ROWS = N * P
BMA = 256
NPAD = ROWS + E * BMA
NB = NPAD // BMA
NBO = 48

NW = 32
TPW = N // NW

_MESH = plsc.VectorSubcoreMesh(
    core_axis_name="c", subcore_axis_name="s", num_cores=2, num_subcores=16
)


def _wid():
    return lax.axis_index("s") * 2 + lax.axis_index("c")


def _router_body(aff_ref, eps_ref, pa_ref, pb_ref, g1_ref, g2_ref, eob_ref):
    aff = aff_ref[...].T
    eps = eps_ref[...].T
    sp = jnp.maximum(aff, 0.0) + jnp.log1p(jnp.exp(-jnp.abs(aff)))
    v = aff + eps * sp

    iota_e = lax.broadcasted_iota(jnp.int32, (E, N), 0)
    cnt = jnp.zeros((E, N), jnp.int32)
    for j in range(E):
        vj = v[j:j + 1]
        beats = jnp.logical_or(
            vj > v, jnp.logical_and(vj == v, j < iota_e))
        cnt = cnt + beats.astype(jnp.int32)
    sel = cnt < TOP_K
    selF = sel.astype(jnp.float32)

    m1 = jnp.max(v, axis=0, keepdims=True)
    ex = jnp.where(sel, jnp.exp(v - m1), 0.0)
    g = ex / jnp.sum(ex, axis=0, keepdims=True)

    m8 = (lax.broadcasted_iota(jnp.int32, (E, E), 1)
          < lax.broadcasted_iota(jnp.int32, (E, E), 0)).astype(jnp.float32)
    seen = jnp.dot(m8, selF, preferred_element_type=jnp.float32)
    first = jnp.logical_and(sel, seen == 0.0)
    second = jnp.logical_and(sel, seen == 1.0)

    u128 = (lax.broadcasted_iota(jnp.int32, (TPW, TPW), 0)
            < lax.broadcasted_iota(jnp.int32, (TPW, TPW), 1)).astype(jnp.float32)
    sel256 = selF.reshape(E * NW, TPW)
    rank = jnp.dot(sel256, u128, preferred_element_type=jnp.float32)
    rank = rank.reshape(E, N)

    counts = jnp.sum(sel256, axis=1, keepdims=True).reshape(E, NW)
    u32 = (lax.broadcasted_iota(jnp.int32, (NW, NW), 0)
           < lax.broadcasted_iota(jnp.int32, (NW, NW), 1)).astype(jnp.float32)
    off = jnp.dot(counts, u32, preferred_element_type=jnp.float32)
    totals = jnp.sum(counts, axis=1, keepdims=True)
    padded = ((totals.astype(jnp.int32) + (BMA - 1)) >> 8 << 8).astype(jnp.float32)
    s = jnp.dot(m8, padded, preferred_element_type=jnp.float32)
    base = jnp.transpose(s + off).astype(jnp.int32)

    def lanesum(x):
        return jnp.sum(x, axis=0, keepdims=True)

    firstF = first.astype(jnp.float32)
    secondF = second.astype(jnp.float32)
    e1 = lanesum(firstF * iota_e.astype(jnp.float32)).astype(jnp.int32)
    e2 = lanesum(secondF * iota_e.astype(jnp.float32)).astype(jnp.int32)
    g1 = lanesum(g * firstF)
    g2 = lanesum(g * secondF)
    r1 = lanesum(rank * firstF).astype(jnp.int32)
    r2 = lanesum(rank * secondF).astype(jnp.int32)

    e1t = e1.reshape(NW, TPW)
    e2t = e2.reshape(NW, TPW)
    b1 = jnp.zeros((NW, TPW), jnp.int32)
    b2 = jnp.zeros((NW, TPW), jnp.int32)
    for e in range(E):
        be = base[:, e:e + 1]
        b1 = jnp.where(e1t == e, be, b1)
        b2 = jnp.where(e2t == e, be, b2)
    pa_ref[...] = b1 + r1.reshape(NW, TPW)
    pb_ref[...] = b2 + r2.reshape(NW, TPW)
    g1_ref[...] = g1.reshape(NW, TPW)
    g2_ref[...] = g2.reshape(NW, TPW)

    bi = (lax.broadcasted_iota(jnp.int32, (1, NBO), 1) * BMA).astype(jnp.float32)
    eb = jnp.full((1, NBO), -1.0, jnp.float32)
    for e in range(E):
        eb = eb + (bi >= s[e:e + 1, :]).astype(jnp.float32)
    eob_ref[...] = eb.astype(jnp.int32)


def _router(aff, eps):
    return pl.pallas_call(
        _router_body,
        out_shape=(
            jax.ShapeDtypeStruct((NW, TPW), jnp.int32),
            jax.ShapeDtypeStruct((NW, TPW), jnp.int32),
            jax.ShapeDtypeStruct((NW, TPW), jnp.float32),
            jax.ShapeDtypeStruct((NW, TPW), jnp.float32),
            jax.ShapeDtypeStruct((1, NBO), jnp.int32),
        ),
    )(aff, eps)


NCHUNK = 4
CT = TPW // NCHUNK


@functools.partial(
    pl.kernel,
    out_type=jax.ShapeDtypeStruct((NPAD, D2), jnp.float32),
    mesh=_MESH,
    compiler_params=pltpu.CompilerParams(needs_layout_passes=False),
    scratch_types=[
        pltpu.VMEM((NCHUNK, CT), jnp.int32),
        pltpu.VMEM((NCHUNK, CT), jnp.int32),
        pltpu.VMEM((CT, D2), jnp.float32),
        pltpu.VMEM((CT, D2), jnp.float32),
        pltpu.SemaphoreType.DMA,
        pltpu.SemaphoreType.DMA,
        pltpu.SemaphoreType.DMA,
        pltpu.SemaphoreType.DMA,
        pltpu.SemaphoreType.DMA,
        pltpu.SemaphoreType.DMA,
    ],
)
def _dispatch(x2_hbm, pa_hbm, pb_hbm, xs2_hbm,
              pa2v, pb2v, buf0, buf1, sl0, sl1, sa0, sa1, sb0, sb1):
    wid = _wid()
    tok0 = wid * TPW
    for c in range(NCHUNK):
        pltpu.sync_copy(pa_hbm.at[wid, pl.ds(c * CT, CT)], pa2v.at[c])
        pltpu.sync_copy(pb_hbm.at[wid, pl.ds(c * CT, CT)], pb2v.at[c])
    bufs = (buf0, buf1)
    sls = (sl0, sl1)
    sas = (sa0, sa1)
    sbs = (sb0, sb1)
    ld = [None, None]
    sca = [None, None]
    scb = [None, None]
    ld[0] = pltpu.async_copy(x2_hbm.at[pl.ds(tok0, CT)], bufs[0], sls[0])
    for c in range(NCHUNK):
        b = c % 2
        nb = (c + 1) % 2
        if c + 1 < NCHUNK:
            if sca[nb] is not None:
                sca[nb].wait()
                scb[nb].wait()
            ld[nb] = pltpu.async_copy(
                x2_hbm.at[pl.ds(tok0 + (c + 1) * CT, CT)], bufs[nb], sls[nb])
        ld[b].wait()
        sca[b] = pltpu.async_copy(bufs[b], xs2_hbm.at[pa2v.at[c]], sas[b])
        scb[b] = pltpu.async_copy(bufs[b], xs2_hbm.at[pb2v.at[c]], sbs[b])
    for b in range(2):
        if sca[b] is not None:
            sca[b].wait()
            scb[b].wait()


def _gmm_body(eob_ref, xs_ref, w_ref, b_ref, out_ref):
    i = pl.program_id(0)
    e = eob_ref[i]
    x = xs_ref[...].astype(jnp.bfloat16)
    w = w_ref[e]
    bias = b_ref[e, 0][None, :]
    ya = jnp.dot(x[:, :D], w, preferred_element_type=jnp.float32) + bias
    yb = jnp.dot(x[:, D:], w, preferred_element_type=jnp.float32) + bias
    out_ref[...] = jnp.concatenate([ya, yb], axis=1)


def _gmm(eob, xs, w, b):
    grid_spec = pltpu.PrefetchScalarGridSpec(
        num_scalar_prefetch=1,
        grid=(NB,),
        in_specs=[
            pl.BlockSpec((BMA, D2), lambda i, eob: (i, 0)),
            pl.BlockSpec((E, D, D), lambda i, eob: (0, 0, 0)),
            pl.BlockSpec((E, 1, D), lambda i, eob: (0, 0, 0)),
        ],
        out_specs=pl.BlockSpec((BMA, D2), lambda i, eob: (i, 0)),
    )
    return pl.pallas_call(
        _gmm_body,
        grid_spec=grid_spec,
        out_shape=jax.ShapeDtypeStruct((NPAD, D2), jnp.float32),
    )(eob, xs, w, b.reshape(E, 1, D))


KCH = 8
KT = TPW // KCH


@functools.partial(
    pl.kernel,
    out_type=jax.ShapeDtypeStruct((N, P, D), jnp.float32),
    mesh=_MESH,
    compiler_params=pltpu.CompilerParams(needs_layout_passes=False),
    scratch_types=[
        pltpu.VMEM((KCH, KT), jnp.int32),
        pltpu.VMEM((KCH, KT), jnp.int32),
        pltpu.VMEM((TPW,), jnp.float32),
        pltpu.VMEM((TPW,), jnp.float32),
        pltpu.VMEM((KT, D2), jnp.float32),
        pltpu.VMEM((KT, D2), jnp.float32),
        pltpu.VMEM((KT, D2), jnp.float32),
        pltpu.VMEM((KT, D2), jnp.float32),
        pltpu.VMEM((KT, D2), jnp.float32),
        pltpu.SemaphoreType.DMA,
        pltpu.SemaphoreType.DMA,
        pltpu.SemaphoreType.DMA,
        pltpu.SemaphoreType.DMA,
        pltpu.SemaphoreType.DMA,
    ],
)
def _combine(ys2_hbm, pa_hbm, pb_hbm, g1_hbm, g2_hbm, out_hbm,
             pa2v, pb2v, g1v, g2v, a0, b0, a1, b1, ob,
             sga0, sgb0, sga1, sgb1, so):
    wid = _wid()
    tok0 = wid * TPW
    pltpu.sync_copy(g1_hbm.at[wid], g1v)
    pltpu.sync_copy(g2_hbm.at[wid], g2v)
    for c in range(KCH):
        pltpu.sync_copy(pa_hbm.at[wid, pl.ds(c * KT, KT)], pa2v.at[c])
        pltpu.sync_copy(pb_hbm.at[wid, pl.ds(c * KT, KT)], pb2v.at[c])
    abufs = (a0, a1)
    bbufs = (b0, b1)
    sgas = (sga0, sga1)
    sgbs = (sgb0, sgb1)
    ga = [None, None]
    gb = [None, None]
    wo = None
    ga[0] = pltpu.async_copy(ys2_hbm.at[pa2v.at[0]], abufs[0], sgas[0])
    gb[0] = pltpu.async_copy(ys2_hbm.at[pb2v.at[0]], bbufs[0], sgbs[0])
    for c in range(KCH):
        b = c % 2
        nb = (c + 1) % 2
        ga[b].wait()
        gb[b].wait()
        if c + 1 < KCH:
            ga[nb] = pltpu.async_copy(
                ys2_hbm.at[pa2v.at[c + 1]], abufs[nb], sgas[nb])
            gb[nb] = pltpu.async_copy(
                ys2_hbm.at[pb2v.at[c + 1]], bbufs[nb], sgbs[nb])
        A = abufs[b]
        B = bbufs[b]
        if wo is not None:
            wo[0].wait()
            wo[1].wait()

        def body(l, _):
            gav = plsc.load_gather(g1v, [jnp.full((16,), c * KT, jnp.int32) + l])
            gbv = plsc.load_gather(g2v, [jnp.full((16,), c * KT, jnp.int32) + l])
            for k in range(D2 // 16):
                ks = pl.ds(k * 16, 16)
                ob[l, ks] = A[l, ks] * gav + B[l, ks] * gbv
            return 0

        lax.fori_loop(0, KT, body, 0)
        osl = pl.ds(tok0 + c * KT, KT)
        wo = (
            pltpu.async_copy(ob.at[:, pl.ds(0, D)], out_hbm.at[osl, 0], so),
            pltpu.async_copy(ob.at[:, pl.ds(D, D)], out_hbm.at[osl, 1], so),
        )
    wo[0].wait()
    wo[1].wait()


def kernel(patch_x, patch_embedding, affinity, noise_eps, expert_W, expert_b):
    x2 = patch_x.reshape(N, D2)
    pa, pb, g1, g2, eob = _router(affinity, noise_eps)
    xs2 = _dispatch(x2, pa, pb)
    ys2 = _gmm(eob.reshape(NBO), xs2, expert_W.astype(jnp.bfloat16), expert_b)
    return _combine(ys2, pa, pb, g1, g2)


# speedup vs baseline: 1.0235x; 1.0071x over previous
import functools

import jax
import jax.numpy as jnp
from jax import lax
from jax.experimental import pallas as pl
from jax.experimental.pallas import tpu as pltpu
from jax.experimental.pallas import tpu_sc as plsc

TOP_K = 2
E = 8
D = 768
N = 4096
P = 2
D2 = D * P
ROWS = N * P
BMA = 256
NPAD = ROWS + E * BMA
NB = NPAD // BMA
NBO = 48

NW = 32
TPW = N // NW

_MESH = plsc.VectorSubcoreMesh(
    core_axis_name="c", subcore_axis_name="s", num_cores=2, num_subcores=16
)


def _wid():
    return lax.axis_index("s") * 2 + lax.axis_index("c")


def _router_body(aff_ref, eps_ref, pa_ref, pb_ref, g1_ref, g2_ref, eob_ref):
    aff = aff_ref[...].T
    eps = eps_ref[...].T
    sp = jnp.maximum(aff, 0.0) + jnp.log1p(jnp.exp(-jnp.abs(aff)))
    v = aff + eps * sp

    iota_e = lax.broadcasted_iota(jnp.int32, (E, N), 0)
    cnt = jnp.zeros((E, N), jnp.int32)
    for j in range(E):
        vj = v[j:j + 1]
        beats = jnp.logical_or(
            vj > v, jnp.logical_and(vj == v, j < iota_e))
        cnt = cnt + beats.astype(jnp.int32)
    sel = cnt < TOP_K
    selF = sel.astype(jnp.float32)

    m1 = jnp.max(v, axis=0, keepdims=True)
    ex = jnp.where(sel, jnp.exp(v - m1), 0.0)
    g = ex / jnp.sum(ex, axis=0, keepdims=True)

    m8 = (lax.broadcasted_iota(jnp.int32, (E, E), 1)
          < lax.broadcasted_iota(jnp.int32, (E, E), 0)).astype(jnp.float32)
    seen = jnp.dot(m8, selF, preferred_element_type=jnp.float32)
    first = jnp.logical_and(sel, seen == 0.0)
    second = jnp.logical_and(sel, seen == 1.0)

    u128 = (lax.broadcasted_iota(jnp.int32, (TPW, TPW), 0)
            < lax.broadcasted_iota(jnp.int32, (TPW, TPW), 1)).astype(jnp.float32)
    sel256 = selF.reshape(E * NW, TPW)
    rank = jnp.dot(sel256, u128, preferred_element_type=jnp.float32)
    rank = rank.reshape(E, N)

    counts = jnp.sum(sel256, axis=1, keepdims=True).reshape(E, NW)
    u32 = (lax.broadcasted_iota(jnp.int32, (NW, NW), 0)
           < lax.broadcasted_iota(jnp.int32, (NW, NW), 1)).astype(jnp.float32)
    off = jnp.dot(counts, u32, preferred_element_type=jnp.float32)
    totals = jnp.sum(counts, axis=1, keepdims=True)
    padded = ((totals.astype(jnp.int32) + (BMA - 1)) >> 8 << 8).astype(jnp.float32)
    s = jnp.dot(m8, padded, preferred_element_type=jnp.float32)
    base = jnp.transpose(s + off).astype(jnp.int32)

    def lanesum(x):
        return jnp.sum(x, axis=0, keepdims=True)

    firstF = first.astype(jnp.float32)
    secondF = second.astype(jnp.float32)
    e1 = lanesum(firstF * iota_e.astype(jnp.float32)).astype(jnp.int32)
    e2 = lanesum(secondF * iota_e.astype(jnp.float32)).astype(jnp.int32)
    g1 = lanesum(g * firstF)
    g2 = lanesum(g * secondF)
    r1 = lanesum(rank * firstF).astype(jnp.int32)
    r2 = lanesum(rank * secondF).astype(jnp.int32)

    e1t = e1.reshape(NW, TPW)
    e2t = e2.reshape(NW, TPW)
    b1 = jnp.zeros((NW, TPW), jnp.int32)
    b2 = jnp.zeros((NW, TPW), jnp.int32)
    for e in range(E):
        be = base[:, e:e + 1]
        b1 = jnp.where(e1t == e, be, b1)
        b2 = jnp.where(e2t == e, be, b2)
    pa_ref[...] = b1 + r1.reshape(NW, TPW)
    pb_ref[...] = b2 + r2.reshape(NW, TPW)
    g1_ref[...] = g1.reshape(NW, TPW)
    g2_ref[...] = g2.reshape(NW, TPW)

    bi = (lax.broadcasted_iota(jnp.int32, (1, NBO), 1) * BMA).astype(jnp.float32)
    eb = jnp.full((1, NBO), -1.0, jnp.float32)
    for e in range(E):
        eb = eb + (bi >= s[e:e + 1, :]).astype(jnp.float32)
    eob_ref[...] = eb.astype(jnp.int32)


def _router(aff, eps):
    return pl.pallas_call(
        _router_body,
        out_shape=(
            jax.ShapeDtypeStruct((NW, TPW), jnp.int32),
            jax.ShapeDtypeStruct((NW, TPW), jnp.int32),
            jax.ShapeDtypeStruct((NW, TPW), jnp.float32),
            jax.ShapeDtypeStruct((NW, TPW), jnp.float32),
            jax.ShapeDtypeStruct((1, NBO), jnp.int32),
        ),
    )(aff, eps)


NCHUNK = 4
CT = TPW // NCHUNK


@functools.partial(
    pl.kernel,
    out_type=jax.ShapeDtypeStruct((NPAD, D2), jnp.float32),
    mesh=_MESH,
    compiler_params=pltpu.CompilerParams(needs_layout_passes=False),
    scratch_types=[
        pltpu.VMEM((NCHUNK, CT), jnp.int32),
        pltpu.VMEM((NCHUNK, CT), jnp.int32),
        pltpu.VMEM((CT, D2), jnp.float32),
        pltpu.VMEM((CT, D2), jnp.float32),
        pltpu.SemaphoreType.DMA,
        pltpu.SemaphoreType.DMA,
        pltpu.SemaphoreType.DMA,
        pltpu.SemaphoreType.DMA,
        pltpu.SemaphoreType.DMA,
        pltpu.SemaphoreType.DMA,
    ],
)
def _dispatch(x2_hbm, pa_hbm, pb_hbm, xs2_hbm,
              pa2v, pb2v, buf0, buf1, sl0, sl1, sa0, sa1, sb0, sb1):
    wid = _wid()
    tok0 = wid * TPW
    for c in range(NCHUNK):
        pltpu.sync_copy(pa_hbm.at[wid, pl.ds(c * CT, CT)], pa2v.at[c])
        pltpu.sync_copy(pb_hbm.at[wid, pl.ds(c * CT, CT)], pb2v.at[c])
    bufs = (buf0, buf1)
    sls = (sl0, sl1)
    sas = (sa0, sa1)
    sbs = (sb0, sb1)
    ld = [None, None]
    sca = [None, None]
    scb = [None, None]
    ld[0] = pltpu.async_copy(x2_hbm.at[pl.ds(tok0, CT)], bufs[0], sls[0])
    for c in range(NCHUNK):
        b = c % 2
        nb = (c + 1) % 2
        if c + 1 < NCHUNK:
            if sca[nb] is not None:
                sca[nb].wait()
                scb[nb].wait()
            ld[nb] = pltpu.async_copy(
                x2_hbm.at[pl.ds(tok0 + (c + 1) * CT, CT)], bufs[nb], sls[nb])
        ld[b].wait()
        sca[b] = pltpu.async_copy(bufs[b], xs2_hbm.at[pa2v.at[c]], sas[b])
        scb[b] = pltpu.async_copy(bufs[b], xs2_hbm.at[pb2v.at[c]], sbs[b])
    for b in range(2):
        if sca[b] is not None:
            sca[b].wait()
            scb[b].wait()


def _gmm_body(eob_ref, xs_ref, w_ref, b_ref, out_ref):
    i = pl.program_id(0)
    e = eob_ref[i]
    x = xs_ref[...]
    w = w_ref[e]
    bias = b_ref[e, 0][None, :]
    ya = jnp.dot(x[:, :D], w, preferred_element_type=jnp.float32) + bias
    yb = jnp.dot(x[:, D:], w, preferred_element_type=jnp.float32) + bias
    out_ref[...] = jnp.concatenate([ya, yb], axis=1)


def _gmm(eob, xs, w, b):
    grid_spec = pltpu.PrefetchScalarGridSpec(
        num_scalar_prefetch=1,
        grid=(NB,),
        in_specs=[
            pl.BlockSpec((BMA, D2), lambda i, eob: (i, 0)),
            pl.BlockSpec((E, D, D), lambda i, eob: (0, 0, 0)),
            pl.BlockSpec((E, 1, D), lambda i, eob: (0, 0, 0)),
        ],
        out_specs=pl.BlockSpec((BMA, D2), lambda i, eob: (i, 0)),
    )
    return pl.pallas_call(
        _gmm_body,
        grid_spec=grid_spec,
        out_shape=jax.ShapeDtypeStruct((NPAD, D2), jnp.float32),
    )(eob, xs, w, b.reshape(E, 1, D))


KCH = 8
KT = TPW // KCH


@functools.partial(
    pl.kernel,
    out_type=jax.ShapeDtypeStruct((N, P, D), jnp.float32),
    mesh=_MESH,
    compiler_params=pltpu.CompilerParams(needs_layout_passes=False),
    scratch_types=[
        pltpu.VMEM((KCH, KT), jnp.int32),
        pltpu.VMEM((KCH, KT), jnp.int32),
        pltpu.VMEM((TPW,), jnp.float32),
        pltpu.VMEM((TPW,), jnp.float32),
        pltpu.VMEM((KT, D2), jnp.float32),
        pltpu.VMEM((KT, D2), jnp.float32),
        pltpu.VMEM((KT, D2), jnp.float32),
        pltpu.VMEM((KT, D2), jnp.float32),
        pltpu.VMEM((KT, D2), jnp.float32),
        pltpu.SemaphoreType.DMA,
        pltpu.SemaphoreType.DMA,
        pltpu.SemaphoreType.DMA,
        pltpu.SemaphoreType.DMA,
        pltpu.SemaphoreType.DMA,
    ],
)
def _combine(ys2_hbm, pa_hbm, pb_hbm, g1_hbm, g2_hbm, out_hbm,
             pa2v, pb2v, g1v, g2v, a0, b0, a1, b1, ob,
             sga0, sgb0, sga1, sgb1, so):
    wid = _wid()
    tok0 = wid * TPW
    pltpu.sync_copy(g1_hbm.at[wid], g1v)
    pltpu.sync_copy(g2_hbm.at[wid], g2v)
    for c in range(KCH):
        pltpu.sync_copy(pa_hbm.at[wid, pl.ds(c * KT, KT)], pa2v.at[c])
        pltpu.sync_copy(pb_hbm.at[wid, pl.ds(c * KT, KT)], pb2v.at[c])
    abufs = (a0, a1)
    bbufs = (b0, b1)
    sgas = (sga0, sga1)
    sgbs = (sgb0, sgb1)
    ga = [None, None]
    gb = [None, None]
    wo = None
    ga[0] = pltpu.async_copy(ys2_hbm.at[pa2v.at[0]], abufs[0], sgas[0])
    gb[0] = pltpu.async_copy(ys2_hbm.at[pb2v.at[0]], bbufs[0], sgbs[0])
    for c in range(KCH):
        b = c % 2
        nb = (c + 1) % 2
        ga[b].wait()
        gb[b].wait()
        if c + 1 < KCH:
            ga[nb] = pltpu.async_copy(
                ys2_hbm.at[pa2v.at[c + 1]], abufs[nb], sgas[nb])
            gb[nb] = pltpu.async_copy(
                ys2_hbm.at[pb2v.at[c + 1]], bbufs[nb], sgbs[nb])
        A = abufs[b]
        B = bbufs[b]
        if wo is not None:
            wo[0].wait()
            wo[1].wait()

        def body(l, _):
            gav = plsc.load_gather(g1v, [jnp.full((16,), c * KT, jnp.int32) + l])
            gbv = plsc.load_gather(g2v, [jnp.full((16,), c * KT, jnp.int32) + l])
            for k in range(D2 // 16):
                ks = pl.ds(k * 16, 16)
                ob[l, ks] = A[l, ks] * gav + B[l, ks] * gbv
            return 0

        lax.fori_loop(0, KT, body, 0)
        osl = pl.ds(tok0 + c * KT, KT)
        wo = (
            pltpu.async_copy(ob.at[:, pl.ds(0, D)], out_hbm.at[osl, 0], so),
            pltpu.async_copy(ob.at[:, pl.ds(D, D)], out_hbm.at[osl, 1], so),
        )
    wo[0].wait()
    wo[1].wait()


def kernel(patch_x, patch_embedding, affinity, noise_eps, expert_W, expert_b):
    x2 = patch_x.reshape(N, D2)
    pa, pb, g1, g2, eob = _router(affinity, noise_eps)
    xs2 = _dispatch(x2, pa, pb)
    ys2 = _gmm(eob.reshape(NBO), xs2, expert_W, expert_b)
    return _combine(ys2, pa, pb, g1, g2)
